# baseline (speedup 1.0000x reference)
import jax
import jax.numpy as jnp
from jax.experimental import pallas as pl
from jax.experimental.pallas import tpu as pltpu

N = 2048
C = 768
NH = 12
HD = 64
K1 = 614
JB = 512
NJ = N // JB
SCALE = 0.125


def _attnez_body(k_ref, q_ref, e_ref, z_ref):
    kk = k_ref[0]
    qq = q_ref[0]
    s = jax.lax.dot_general(kk, qq, (((1,), (1,)), ((), ())),
                            preferred_element_type=jnp.float32)
    s = s * SCALE
    m = jnp.max(s, axis=0, keepdims=True)
    eT = jnp.exp(s - m)
    e_ref[0] = eT

    def body(tt, acc):
        return acc + e_ref[0, pl.ds(8 * tt, 8), :]

    acc = jax.lax.fori_loop(1, N // 8, body, eT[0:8, :])
    b = acc[0:4, :] + acc[4:8, :]
    c = b[0:2, :] + b[2:4, :]
    z_ref[0] = c[0:1, :] + c[1:2, :]


def _apply_body(e_ref, z_ref, cm_ref, rm_ref, v_ref, out_ref):
    jr = pl.program_id(1)
    eT = e_ref[0]
    z = z_ref[0]
    att = eT / z
    cm = cm_ref[0]
    rm = rm_ref[0]
    mm = cm + rm
    mm = jnp.where(mm == 2.0, 1.0, mm)
    sa = att * mm
    part = jax.lax.dot_general(
        sa.astype(jnp.bfloat16), v_ref[0], (((0,), (0,)), ((), ())),
        preferred_element_type=jnp.float32)

    @pl.when(jr == 0)
    def _():
        out_ref[0] = part

    @pl.when(jr != 0)
    def _():
        out_ref[0] = out_ref[0] + part


def _outproj_body(x_ref, w_ref, b_ref, o_ref):
    o_ref[...] = (jax.lax.dot_general(
        x_ref[...], w_ref[...].astype(jnp.bfloat16), (((1,), (0,)), ((), ())),
        preferred_element_type=jnp.float32) + b_ref[...])


_attnez = pl.pallas_call(
    _attnez_body,
    grid=(NH,),
    in_specs=[
        pl.BlockSpec((1, N, HD), lambda h: (h, 0, 0)),
        pl.BlockSpec((1, N, HD), lambda h: (h, 0, 0)),
    ],
    out_specs=[
        pl.BlockSpec((1, N, N), lambda h: (h, 0, 0)),
        pl.BlockSpec((1, 1, N), lambda h: (h, 0, 0)),
    ],
    out_shape=[
        jax.ShapeDtypeStruct((NH, N, N), jnp.float32),
        jax.ShapeDtypeStruct((NH, 1, N), jnp.float32),
    ],
    compiler_params=pltpu.CompilerParams(dimension_semantics=("arbitrary",)),
)

_apply = pl.pallas_call(
    _apply_body,
    grid=(NH, NJ),
    in_specs=[
        pl.BlockSpec((1, JB, N), lambda h, jr: (h, jr, 0)),
        pl.BlockSpec((1, 1, N), lambda h, jr: (h, 0, 0)),
        pl.BlockSpec((1, JB, 1), lambda h, jr: (h, jr, 0)),
        pl.BlockSpec((1, 1, N), lambda h, jr: (h, 0, 0)),
        pl.BlockSpec((1, JB, HD), lambda h, jr: (h, jr, 0)),
    ],
    out_specs=pl.BlockSpec((1, N, HD), lambda h, jr: (h, 0, 0)),
    out_shape=jax.ShapeDtypeStruct((NH, N, HD), jnp.float32),
    compiler_params=pltpu.CompilerParams(
        dimension_semantics=("arbitrary", "arbitrary")),
)

_outproj = pl.pallas_call(
    _outproj_body,
    out_shape=jax.ShapeDtypeStruct((N, C), jnp.float32),
)


def kernel(x, Wq, Wk, Wv, Wo, bo):
    b, n, c = x.shape
    h = NH

    def split_heads(t):
        return t.reshape(b, n, h, -1).transpose(0, 2, 1, 3).reshape(b * h, n, -1)

    qh = split_heads(x @ Wq).astype(jnp.bfloat16)
    kh = split_heads(x @ Wk).astype(jnp.bfloat16)
    vh = split_heads(x @ Wv).astype(jnp.bfloat16)

    e, z = _attnez(kh, qh)
    z2 = z.reshape(h, n)

    attn = e / z2[:, None, :]
    k1 = K1
    bh = b * h
    row_variance = attn.mean(axis=2)
    _, idx2 = jax.lax.top_k(row_variance, k1)
    row_variance1 = attn.mean(axis=1)
    _, idx = jax.lax.top_k(row_variance1, k1)
    bsel = jnp.arange(bh)[:, None]
    colmask = jnp.zeros((bh, n), dtype=attn.dtype).at[bsel, idx2].set(1.0)
    rowmask = jnp.zeros((bh, n), dtype=attn.dtype).at[bsel, idx].set(1.0)
    MmatT = colmask[:, :, None] + rowmask[:, None, :]
    MmatT = jnp.where(MmatT == 2.0, 1.0, MmatT)
    sparse_attnT = attn * MmatT
    A = sparse_attnT.reshape(b, h, n, n).mean(axis=1)
    A_mean = A.mean(axis=2)
    _, loc = jax.lax.top_k(A_mean, k1)

    out_h = _apply(e, z, colmask.reshape(h, n, 1),
                   rowmask.reshape(h, 1, n), vh)

    out2d = out_h.astype(jnp.bfloat16).transpose(1, 0, 2).reshape(n, h * HD)
    f2d = _outproj(out2d, Wo, bo.reshape(1, C))
    feature = f2d.reshape(1, n, C).transpose(0, 2, 1)
    return feature, loc


# speedup vs baseline: 1.0108x; 1.0108x over previous
import jax
import jax.numpy as jnp
from jax.experimental import pallas as pl
from jax.experimental.pallas import tpu as pltpu

N = 2048
C = 768
NH = 12
HD = 64
K1 = 614
JB = 512
NJ = N // JB
SCALE = 0.125


def _attnez_body(k_ref, q_ref, e_ref, z_ref):
    kk = k_ref[0]
    qq = q_ref[0]
    s = jax.lax.dot_general(kk, qq, (((1,), (1,)), ((), ())),
                            preferred_element_type=jnp.float32)
    s = s * SCALE
    m = jnp.max(s, axis=0, keepdims=True)
    eT = jnp.exp(s - m)
    e_ref[0] = eT

    def body(tt, acc):
        return acc + e_ref[0, pl.ds(8 * tt, 8), :]

    acc = jax.lax.fori_loop(1, N // 8, body, eT[0:8, :])
    b = acc[0:4, :] + acc[4:8, :]
    c = b[0:2, :] + b[2:4, :]
    z_ref[0] = c[0:1, :] + c[1:2, :]


def _apply_body(e_ref, z_ref, cm_ref, rm_ref, v_ref, wo_ref, b_ref, f_ref,
                acc_ref):
    h = pl.program_id(0)
    jr = pl.program_id(1)
    eT = e_ref[0]
    z = z_ref[0]
    att = eT / z
    cm = cm_ref[0]
    rm = rm_ref[0]
    mm = cm + rm
    mm = jnp.where(mm == 2.0, 1.0, mm)
    sa = att * mm
    part = jax.lax.dot_general(
        sa.astype(jnp.bfloat16), v_ref[0], (((0,), (0,)), ((), ())),
        preferred_element_type=jnp.float32)

    @pl.when(jr == 0)
    def _():
        acc_ref[...] = part

    @pl.when(jr != 0)
    def _():
        acc_ref[...] = acc_ref[...] + part

    @pl.when(jr == NJ - 1)
    def _():
        fo = jax.lax.dot_general(
            acc_ref[...].astype(jnp.bfloat16), wo_ref[0],
            (((1,), (0,)), ((), ())),
            preferred_element_type=jnp.float32)

        @pl.when(h == 0)
        def _():
            f_ref[...] = fo + b_ref[...]

        @pl.when(h != 0)
        def _():
            f_ref[...] = f_ref[...] + fo


_attnez = pl.pallas_call(
    _attnez_body,
    grid=(NH,),
    in_specs=[
        pl.BlockSpec((1, N, HD), lambda h: (h, 0, 0)),
        pl.BlockSpec((1, N, HD), lambda h: (h, 0, 0)),
    ],
    out_specs=[
        pl.BlockSpec((1, N, N), lambda h: (h, 0, 0)),
        pl.BlockSpec((1, 1, N), lambda h: (h, 0, 0)),
    ],
    out_shape=[
        jax.ShapeDtypeStruct((NH, N, N), jnp.float32),
        jax.ShapeDtypeStruct((NH, 1, N), jnp.float32),
    ],
    compiler_params=pltpu.CompilerParams(dimension_semantics=("arbitrary",)),
)

_apply = pl.pallas_call(
    _apply_body,
    grid=(NH, NJ),
    in_specs=[
        pl.BlockSpec((1, JB, N), lambda h, jr: (h, jr, 0)),
        pl.BlockSpec((1, 1, N), lambda h, jr: (h, 0, 0)),
        pl.BlockSpec((1, JB, 1), lambda h, jr: (h, jr, 0)),
        pl.BlockSpec((1, 1, N), lambda h, jr: (h, 0, 0)),
        pl.BlockSpec((1, JB, HD), lambda h, jr: (h, jr, 0)),
        pl.BlockSpec((1, HD, C), lambda h, jr: (h, 0, 0)),
        pl.BlockSpec((1, C), lambda h, jr: (0, 0)),
    ],
    out_specs=pl.BlockSpec((N, C), lambda h, jr: (0, 0)),
    out_shape=jax.ShapeDtypeStruct((N, C), jnp.float32),
    scratch_shapes=[pltpu.VMEM((N, HD), jnp.float32)],
    compiler_params=pltpu.CompilerParams(
        dimension_semantics=("arbitrary", "arbitrary")),
)


def kernel(x, Wq, Wk, Wv, Wo, bo):
    b, n, c = x.shape
    h = NH

    def split_heads(t):
        return t.reshape(b, n, h, -1).transpose(0, 2, 1, 3).reshape(b * h, n, -1)

    qh = split_heads(x @ Wq).astype(jnp.bfloat16)
    kh = split_heads(x @ Wk).astype(jnp.bfloat16)
    vh = split_heads(x @ Wv).astype(jnp.bfloat16)

    e, z = _attnez(kh, qh)
    z2 = z.reshape(h, n)

    attn = e / z2[:, None, :]
    k1 = K1
    bh = b * h
    row_variance = attn.mean(axis=2)
    _, idx2 = jax.lax.top_k(row_variance, k1)
    row_variance1 = attn.mean(axis=1)
    _, idx = jax.lax.top_k(row_variance1, k1)
    bsel = jnp.arange(bh)[:, None]
    colmask = jnp.zeros((bh, n), dtype=attn.dtype).at[bsel, idx2].set(1.0)
    rowmask = jnp.zeros((bh, n), dtype=attn.dtype).at[bsel, idx].set(1.0)
    MmatT = colmask[:, :, None] + rowmask[:, None, :]
    MmatT = jnp.where(MmatT == 2.0, 1.0, MmatT)
    sparse_attnT = attn * MmatT
    A = sparse_attnT.reshape(b, h, n, n).mean(axis=1)
    A_mean = A.mean(axis=2)
    _, loc = jax.lax.top_k(A_mean, k1)

    wo_h = Wo.reshape(h, HD, C).astype(jnp.bfloat16)
    f2d = _apply(e, z, colmask.reshape(h, n, 1),
                 rowmask.reshape(h, 1, n), vh, wo_h, bo.reshape(1, C))
    feature = f2d.reshape(1, n, C).transpose(0, 2, 1)
    return feature, loc
